# baseline (speedup 1.0000x reference)
import functools

import jax
import jax.numpy as jnp
from jax import lax
from jax.experimental import pallas as pl
from jax.experimental.pallas import tpu as pltpu
from jax.experimental.pallas import tpu_sc as plsc

NUM_EMBEDDINGS = 1000000
EMBED_DIM = 64
BATCH = 16384
N_FIELDS = 26
TOTAL = BATCH * N_FIELDS

NUM_CORES = 2
NUM_SUBCORES = 16
NUM_WORKERS = NUM_CORES * NUM_SUBCORES

PER_WORKER = TOTAL // NUM_WORKERS
CHUNK = 128
N_CHUNKS = PER_WORKER // CHUNK

assert PER_WORKER * NUM_WORKERS == TOTAL
assert N_CHUNKS * CHUNK == PER_WORKER
assert N_CHUNKS % 2 == 0


def _gather_body(idx_hbm, table_hbm, out_hbm, idx_v, rows0, rows1, sem0, sem1):
    wid = lax.axis_index("s") * NUM_CORES + lax.axis_index("c")
    base = wid * PER_WORKER

    pltpu.sync_copy(idx_hbm.at[wid], idx_v)

    rows = (rows0, rows1)
    sems = (sem0, sem1)

    def start(i, b):
        pltpu.async_copy(table_hbm.at[idx_v.at[i]], rows[b], sems[b])

    def finish(i, b):
        pltpu.make_async_copy(table_hbm.at[idx_v.at[i]], rows[b], sems[b]).wait()
        pltpu.sync_copy(rows[b], out_hbm.at[pl.ds(base + i * CHUNK, CHUNK)])

    start(0, 0)
    start(1, 1)

    def step(g, carry):
        i0 = 2 * g
        finish(i0, 0)
        start(i0 + 2, 0)
        finish(i0 + 1, 1)
        start(i0 + 3, 1)
        return carry

    lax.fori_loop(0, N_CHUNKS // 2 - 1, step, 0)

    finish(N_CHUNKS - 2, 0)
    finish(N_CHUNKS - 1, 1)


_gather_call = functools.partial(
    pl.kernel,
    mesh=plsc.VectorSubcoreMesh(core_axis_name="c", subcore_axis_name="s"),
    out_type=jax.ShapeDtypeStruct((TOTAL, EMBED_DIM), jnp.float32),
    scratch_types=[
        pltpu.VMEM((N_CHUNKS, CHUNK), jnp.int32),
        pltpu.VMEM((CHUNK, EMBED_DIM), jnp.float32),
        pltpu.VMEM((CHUNK, EMBED_DIM), jnp.float32),
        pltpu.SemaphoreType.DMA,
        pltpu.SemaphoreType.DMA,
    ],
    compiler_params=pltpu.CompilerParams(use_tc_tiling_on_sc=False),
)(_gather_body)


@jax.jit
def kernel(indices, W):
    flat = indices.reshape(-1).astype(jnp.int32)
    idx3 = flat.reshape(NUM_WORKERS, N_CHUNKS, CHUNK)
    out = _gather_call(idx3, W)
    return out.reshape(BATCH, N_FIELDS, EMBED_DIM)


# speedup vs baseline: 1.2183x; 1.2183x over previous
import functools

import numpy as np
import jax
import jax.numpy as jnp
from jax import lax
from jax.experimental import pallas as pl
from jax.experimental.pallas import tpu as pltpu
from jax.experimental.pallas import tpu_sc as plsc

NUM_EMBEDDINGS = 1000000
EMBED_DIM = 64
LANE = 128
BATCH = 16384
N_FIELDS = 26
F_PAD = 32
TOTAL = BATCH * N_FIELDS

NUM_CORES = 2
NUM_SUBCORES = 16
NUM_WORKERS = NUM_CORES * NUM_SUBCORES

PER_WORKER = TOTAL // NUM_WORKERS
CHUNK = 128
N_CHUNKS = PER_WORKER // CHUNK
NBUF = 4

assert PER_WORKER * NUM_WORKERS == TOTAL
assert N_CHUNKS * CHUNK == PER_WORKER
assert N_CHUNKS % NBUF == 0

_P = np.arange(TOTAL, dtype=np.int64)
_DEST_ROWS = (F_PAD * (_P // N_FIELDS) + (_P % N_FIELDS)).astype(np.int32)
_DEST3 = _DEST_ROWS.reshape(NUM_WORKERS, N_CHUNKS, CHUNK)


def _gather_body(idx_hbm, dest_hbm, table_hbm, out_hbm,
                 idx_v, dest_v, rows, sg, ss):
    wid = lax.axis_index("s") * NUM_CORES + lax.axis_index("c")

    pltpu.sync_copy(idx_hbm.at[wid], idx_v)
    pltpu.sync_copy(dest_hbm.at[wid], dest_v)

    def start_gather(i, b):
        pltpu.async_copy(table_hbm.at[idx_v.at[i]], rows[b].at[0], sg[b])

    def wait_gather(i, b):
        pltpu.make_async_copy(table_hbm.at[idx_v.at[i]], rows[b].at[0],
                              sg[b]).wait()

    def start_scatter(i, b):
        pltpu.async_copy(rows[b].at[0], out_hbm.at[dest_v.at[i]], ss[b])

    def wait_scatter(i, b):
        pltpu.make_async_copy(rows[b].at[0], out_hbm.at[dest_v.at[i]],
                              ss[b]).wait()

    for b in range(NBUF):
        start_gather(b, b)

    def step(g, carry):
        i0 = g * NBUF
        for b in range(NBUF):
            wait_gather(i0 + b, b)
            start_scatter(i0 + b, b)
        for b in range(NBUF):
            wait_scatter(i0 + b, b)
            start_gather(i0 + NBUF + b, b)
        return carry

    lax.fori_loop(0, N_CHUNKS // NBUF - 1, step, 0)

    i0 = N_CHUNKS - NBUF
    for b in range(NBUF):
        wait_gather(i0 + b, b)
        start_scatter(i0 + b, b)
    for b in range(NBUF):
        wait_scatter(i0 + b, b)


_gather_call = functools.partial(
    pl.kernel,
    mesh=plsc.VectorSubcoreMesh(core_axis_name="c", subcore_axis_name="s"),
    out_type=jax.ShapeDtypeStruct((BATCH * F_PAD, LANE), jnp.float32),
    scratch_types=[
        pltpu.VMEM((N_CHUNKS, CHUNK), jnp.int32),
        pltpu.VMEM((N_CHUNKS, CHUNK), jnp.int32),
    ] + [pltpu.VMEM((1, CHUNK, LANE), jnp.float32) for _ in range(NBUF)]
      + [pltpu.SemaphoreType.DMA for _ in range(2 * NBUF)],
    compiler_params=pltpu.CompilerParams(use_tc_tiling_on_sc=True),
)


def _body_wrap(idx_hbm, dest_hbm, table_hbm, out_hbm, *scratch):
    idx_v, dest_v = scratch[0], scratch[1]
    rows = scratch[2:2 + NBUF]
    sg = scratch[2 + NBUF:2 + 2 * NBUF]
    ss = scratch[2 + 2 * NBUF:]
    _gather_body(idx_hbm, dest_hbm, table_hbm, out_hbm,
                 idx_v, dest_v, rows, sg, ss)


@jax.jit
def kernel(indices, W):
    flat = indices.reshape(-1).astype(jnp.int32)
    idx3 = flat.reshape(NUM_WORKERS, N_CHUNKS, CHUNK)
    dest3 = jnp.asarray(_DEST3)
    W_pad = jnp.pad(W, ((0, 0), (0, LANE - EMBED_DIM)))
    out = _gather_call(_body_wrap)(idx3, dest3, W_pad)
    out4 = out.reshape(BATCH, F_PAD, LANE)
    return out4[:, :N_FIELDS, :EMBED_DIM]
